# baseline (speedup 1.0000x reference)
import functools

import jax
import jax.numpy as jnp
from jax import lax
from jax.experimental import pallas as pl
from jax.experimental.pallas import tpu as pltpu
from jax.experimental.pallas import tpu_sc as plsc

_NW = 32
_DEPTH = 8


def _gather_kernel(bsz, seq, xt_hbm, table_hbm, out_hbm, idx_v, *bufs):
    rows = bufs[0:_DEPTH]
    semg = bufs[_DEPTH:2 * _DEPTH]
    semw = bufs[2 * _DEPTH:3 * _DEPTH]

    wid = lax.axis_index("s") * 2 + lax.axis_index("c")
    bw = bsz // _NW
    b0 = wid * bw

    pltpu.sync_copy(xt_hbm.at[:, pl.ds(b0, bw)], idx_v)

    def g_desc(s, k):
        return pltpu.make_async_copy(table_hbm.at[idx_v.at[s]], rows[k], semg[k])

    def w_desc(s, k):
        return pltpu.make_async_copy(rows[k], out_hbm.at[s, wid], semw[k])

    n_iters = seq // _DEPTH

    for k in range(_DEPTH):
        g_desc(k, k).start()

    def body(j, _):
        s0 = j * _DEPTH
        for k in range(_DEPTH):
            g_desc(s0 + k, k).wait()
            w_desc(s0 + k, k).start()

        @pl.when(j < n_iters - 1)
        def _refill():
            for k in range(_DEPTH):
                w_desc(s0 + k, k).wait()
                g_desc(s0 + _DEPTH + k, k).start()

        return 0

    lax.fori_loop(0, n_iters, body, 0)

    for k in range(_DEPTH):
        w_desc(seq - _DEPTH + k, k).wait()


def kernel(x, embed):
    b, s = x.shape
    v, d = embed.shape
    nb = b // 128

    mesh = plsc.VectorSubcoreMesh(core_axis_name="c", subcore_axis_name="s")

    run = pl.kernel(
        functools.partial(_gather_kernel, b, s),
        mesh=mesh,
        out_type=jax.ShapeDtypeStruct((s, nb, 128, d), jnp.float32),
        scratch_types=(
            [pltpu.VMEM((s, b // _NW), jnp.int32)]
            + [pltpu.VMEM((b // _NW, d), jnp.float32)] * _DEPTH
            + [pltpu.SemaphoreType.DMA] * (2 * _DEPTH)
        ),
        compiler_params=pltpu.CompilerParams(use_tc_tiling_on_sc=False,
                                             needs_layout_passes=False),
    )
    xt = jnp.transpose(x.astype(jnp.int32))
    out4 = run(xt, embed)
    return out4.transpose(1, 2, 0, 3).reshape(b, s, d)
